# baseline (speedup 1.0000x reference)
import functools

import jax
import jax.numpy as jnp
from jax import lax
from jax.experimental import pallas as pl
from jax.experimental.pallas import tpu as pltpu
from jax.experimental.pallas import tpu_sc as plsc

D = 1024
B = 4 * 8192
NC, NS = 2, 16
NW = NC * NS
BPW = B // NW
C = 8
NBUF = 8
NCHUNK = BPW // C

_mesh = plsc.VectorSubcoreMesh(core_axis_name="c", subcore_axis_name="s")


@functools.partial(
    pl.kernel,
    mesh=_mesh,
    out_type=jax.ShapeDtypeStruct((B, D), jnp.float32),
    scratch_types=[
        pltpu.VMEM((BPW,), jnp.int32),
        pltpu.VMEM((NBUF, C, D), jnp.float32),
    ]
    + [pltpu.SemaphoreType.DMA] * (2 * NBUF),
)
def _gather_rows(idx_hbm, table_hbm, out_hbm, idx_v, rows_v, *sems):
    gsems = sems[:NBUF]
    ssems = sems[NBUF:]
    wid = lax.axis_index("s") * NC + lax.axis_index("c")
    base = wid * BPW
    pltpu.sync_copy(idx_hbm.at[pl.ds(base, BPW)], idx_v)

    def g_start(b, i):
        pltpu.async_copy(
            table_hbm.at[idx_v.at[pl.ds(i * C, C)]], rows_v.at[b], gsems[b]
        )

    def g_wait(b):
        pltpu.make_async_copy(
            table_hbm.at[idx_v.at[pl.ds(0, C)]], rows_v.at[b], gsems[b]
        ).wait()

    def s_start(b, i):
        pltpu.async_copy(
            rows_v.at[b], out_hbm.at[pl.ds(base + i * C, C)], ssems[b]
        )

    def s_wait(b):
        pltpu.make_async_copy(
            rows_v.at[b], out_hbm.at[pl.ds(base, C)], ssems[b]
        ).wait()

    for b in range(NBUF):
        g_start(b, b)

    def outer(t, carry):
        gi = t * NBUF
        for b in range(NBUF):
            g_wait(b)
            s_start(b, gi + b)
            if b >= 4:
                s_wait(b - 4)
                g_start(b - 4, gi + NBUF + b - 4)
        for b in range(NBUF - 4, NBUF):
            s_wait(b)
            g_start(b, gi + NBUF + b)
        return carry

    lax.fori_loop(0, NCHUNK // NBUF - 1, outer, 0)
    gi = NCHUNK - NBUF
    for b in range(NBUF):
        g_wait(b)
        s_start(b, gi + b)
    for b in range(NBUF):
        s_wait(b)


def kernel(position_ids, table):
    idx = position_ids.reshape(-1).astype(jnp.int32)
    out = _gather_rows(idx, table)
    return lax.stop_gradient(out.reshape(position_ids.shape + (D,)))


# speedup vs baseline: 1.0092x; 1.0092x over previous
import functools

import jax
import jax.numpy as jnp
from jax import lax
from jax.experimental import pallas as pl
from jax.experimental.pallas import tpu as pltpu
from jax.experimental.pallas import tpu_sc as plsc

D = 1024
B = 4 * 8192
NC, NS = 2, 16
NW = NC * NS
BPW = B // NW
C = 8
NBUF = 8
NCHUNK = BPW // C

_mesh = plsc.VectorSubcoreMesh(core_axis_name="c", subcore_axis_name="s")


@functools.partial(
    pl.kernel,
    mesh=_mesh,
    out_type=jax.ShapeDtypeStruct((B, D), jnp.float32),
    scratch_types=[
        pltpu.VMEM((BPW,), jnp.int32),
        pltpu.VMEM((NBUF, C, D), jnp.float32),
    ]
    + [pltpu.SemaphoreType.DMA] * (2 * NBUF),
)
def _gather_rows(idx_hbm, table_hbm, out_hbm, idx_v, rows_v, *sems):
    gsems = sems[:NBUF]
    ssems = sems[NBUF:]
    wid = lax.axis_index("s") * NC + lax.axis_index("c")
    base = wid * BPW
    pltpu.sync_copy(idx_hbm.at[pl.ds(base, BPW)], idx_v)

    def g_start(b, i):
        pltpu.async_copy(
            table_hbm.at[idx_v.at[pl.ds(i * C, C)]], rows_v.at[b], gsems[b]
        )

    def g_wait(b):
        pltpu.make_async_copy(
            table_hbm.at[idx_v.at[pl.ds(0, C)]], rows_v.at[b], gsems[b]
        ).wait()

    def s_start(b, i):
        pltpu.async_copy(
            rows_v.at[b], out_hbm.at[pl.ds(base + i * C, C)], ssems[b]
        )

    def s_wait(b):
        pltpu.make_async_copy(
            rows_v.at[b], out_hbm.at[pl.ds(base, C)], ssems[b]
        ).wait()

    for b in range(NBUF):
        g_start(b, b)

    def outer(t, carry):
        gi = t * NBUF
        for b in range(NBUF):
            g_wait(b)
            s_start(b, gi + b)
            if b >= 1:
                s_wait(b - 1)
                g_start(b - 1, gi + NBUF + b - 1)
        for b in range(NBUF - 1, NBUF):
            s_wait(b)
            g_start(b, gi + NBUF + b)
        return carry

    lax.fori_loop(0, NCHUNK // NBUF - 1, outer, 0)
    gi = NCHUNK - NBUF
    for b in range(NBUF):
        g_wait(b)
        s_start(b, gi + b)
    for b in range(NBUF):
        s_wait(b)


def kernel(position_ids, table):
    idx = position_ids.reshape(-1).astype(jnp.int32)
    out = _gather_rows(idx, table)
    return lax.stop_gradient(out.reshape(position_ids.shape + (D,)))
